# Initial kernel scaffold; baseline (speedup 1.0000x reference)
import jax
import jax.numpy as jnp
from jax.experimental import pallas as pl


def kernel(s, v, pos, edges, edge_w1, edge_b1, edge_w2, edge_b2, posmlp_w1, posmlp_b1, posmlp_w2, posmlp_b2, vpos_w, vmsg_w1, vmsg_b1, vmsg_w2, vmsg_b2, smlp_w1, smlp_b1, smlp_w2, smlp_b2, gate_w, gate_b):
    raise NotImplementedError("write your pallas kernel here")


# speedup vs baseline: 13.7951x; 13.7951x over previous
import functools

import jax
import jax.numpy as jnp
from jax import lax
from jax.experimental import pallas as pl
from jax.experimental.pallas import tpu as pltpu
from jax.experimental.pallas import tpu_sc as plsc

NC = 2
NS = 16
NW = NC * NS
BLK = 80

WI = 144
WJ = 160

_f32 = jnp.float32


def _silu(x):
    return x * jax.nn.sigmoid(x)


def _tc_pre(s, vT, pos16, w1a, w1b, b1, vm_w1, vm_b1, vm_w2, vm_b2, vposw):
    N, H = s.shape
    BN = 2000

    def body(s_ref, vT_ref, pos_ref, w1a_ref, w1b_ref, b1_ref, vw1_ref, vb1_ref,
             vw2_ref, vb2_ref, vp_ref, tabI_ref, tabJ_ref, vs_ref):
        sb = s_ref[...]
        a = jnp.dot(sb, w1a_ref[...], preferred_element_type=_f32)
        b = jnp.dot(sb, w1b_ref[...], preferred_element_type=_f32) + b1_ref[...]
        p16 = pos_ref[...]
        tabI_ref[...] = jnp.concatenate([a, p16], axis=1)
        t1 = jnp.dot(sb, vw1_ref[...], preferred_element_type=_f32) + vb1_ref[...]
        wv = jnp.dot(_silu(t1), vw2_ref[...], preferred_element_type=_f32) + vb2_ref[...]
        vb = vT_ref[...]
        vs = vb * wv[None]
        vs_ref[...] = vs
        vp = vp_ref[...]
        us = [jnp.sum(vs[t] * vp, axis=1, keepdims=True) for t in range(3)]
        u16 = jnp.concatenate(us + [jnp.zeros((BN, 13), _f32)], axis=1)
        tabJ_ref[...] = jnp.concatenate([b, p16, u16], axis=1)

    grid = (N // BN,)
    full = lambda shape: pl.BlockSpec(shape, lambda n: (0,) * len(shape))
    return pl.pallas_call(
        body,
        grid=grid,
        in_specs=[
            pl.BlockSpec((BN, H), lambda n: (n, 0)),
            pl.BlockSpec((3, BN, H), lambda n: (0, n, 0)),
            pl.BlockSpec((BN, 16), lambda n: (n, 0)),
            full((H, H)), full((H, H)), full((1, H)),
            full((H, H)), full((1, H)), full((H, H)), full((1, H)),
            full((1, H)),
        ],
        out_specs=[
            pl.BlockSpec((BN, WI), lambda n: (n, 0)),
            pl.BlockSpec((BN, WJ), lambda n: (n, 0)),
            pl.BlockSpec((3, BN, H), lambda n: (0, n, 0)),
        ],
        out_shape=[
            jax.ShapeDtypeStruct((N, WI), _f32),
            jax.ShapeDtypeStruct((N, WJ), _f32),
            jax.ShapeDtypeStruct((3, N, H), _f32),
        ],
    )(s, vT, pos16, w1a, w1b, b1, vm_w1, vm_b1, vm_w2, vm_b2, vposw)


def _sc_gather(ei, ej, tabI, tabJ):
    E = ei.shape[0]
    epw = E // NW
    nblk = epw // BLK

    @functools.partial(
        pl.kernel,
        mesh=plsc.VectorSubcoreMesh(core_axis_name="c", subcore_axis_name="s"),
        compiler_params=pltpu.CompilerParams(use_tc_tiling_on_sc=False),
        out_type=[
            jax.ShapeDtypeStruct((E, WI), _f32),
            jax.ShapeDtypeStruct((E, WJ), _f32),
        ],
        scratch_types=[
            pltpu.VMEM((BLK,), jnp.int32),
            pltpu.VMEM((BLK,), jnp.int32),
            pltpu.VMEM((BLK, WI), _f32),
            pltpu.VMEM((BLK, WJ), _f32),
            pltpu.SemaphoreType.DMA,
            pltpu.SemaphoreType.DMA,
        ],
    )
    def k(ei_h, ej_h, tabI_h, tabJ_h, gI_h, gJ_h, idxi, idxj, ri, rj, s1, s2):
        c = lax.axis_index("c")
        s = lax.axis_index("s")
        base = (s * NC + c) * epw

        def body(b, carry):
            off = base + b * BLK
            pltpu.sync_copy(ei_h.at[pl.ds(off, BLK)], idxi)
            pltpu.sync_copy(ej_h.at[pl.ds(off, BLK)], idxj)
            cp1 = pltpu.async_copy(tabI_h.at[idxi], ri, s1)
            cp2 = pltpu.async_copy(tabJ_h.at[idxj], rj, s2)
            cp1.wait()
            cp2.wait()
            pltpu.sync_copy(ri, gI_h.at[pl.ds(off, BLK)])
            pltpu.sync_copy(rj, gJ_h.at[pl.ds(off, BLK)])
            return carry

        lax.fori_loop(0, nblk, body, 0)

    return k(ei, ej, tabI, tabJ)


def _tc_edge(gI, gJ, w1c, Wep, bep, pw2, pb2):
    E = gI.shape[0]
    BE = 2000

    def body(gI_ref, gJ_ref, w1c_ref, wep_ref, bep_ref, pw2_ref, pb2_ref,
             h_ref, pc_ref):
        gI = gI_ref[...]
        gJ = gJ_ref[...]
        r = gI[:, 128:131] - gJ[:, 128:131]
        dist_sq = jnp.sum(r * r, axis=1, keepdims=True)
        dist = jnp.sqrt(dist_sq + 1e-6)
        dirij = r / (dist + 1e-8)
        pre = gI[:, :128] + gJ[:, :128] + dist_sq * w1c_ref[...]
        h = pre * jax.nn.sigmoid(pre)
        h_ref[...] = h
        t1 = jnp.dot(h, wep_ref[...], preferred_element_type=_f32) + bep_ref[...]
        pw = jnp.dot(_silu(t1), pw2_ref[...], preferred_element_type=_f32) + pb2_ref[0, 0]
        pmsg = pw * dirij + gJ[:, 144:147]
        pc_ref[...] = jnp.concatenate(
            [pmsg, jnp.ones((BE, 1), _f32), jnp.zeros((BE, 12), _f32)], axis=1)

    full = lambda shape: pl.BlockSpec(shape, lambda n: (0,) * len(shape))
    return pl.pallas_call(
        body,
        grid=(E // BE,),
        in_specs=[
            pl.BlockSpec((BE, WI), lambda n: (n, 0)),
            pl.BlockSpec((BE, WJ), lambda n: (n, 0)),
            full((1, 128)), full((128, 128)), full((1, 128)),
            full((128, 1)), full((1, 1)),
        ],
        out_specs=[
            pl.BlockSpec((BE, 128), lambda n: (n, 0)),
            pl.BlockSpec((BE, 16), lambda n: (n, 0)),
        ],
        out_shape=[
            jax.ShapeDtypeStruct((E, 128), _f32),
            jax.ShapeDtypeStruct((E, 16), _f32),
        ],
    )(gI, gJ, w1c, Wep, bep, pw2, pb2)


def _sc_scatter_hp(ei, h, pc, zeros128, zeros16):
    E = ei.shape[0]
    N = zeros128.shape[0]
    epw = E // NW
    nblk = epw // BLK
    rps = N // NS

    @functools.partial(
        pl.kernel,
        mesh=plsc.VectorSubcoreMesh(core_axis_name="c", subcore_axis_name="s"),
        compiler_params=pltpu.CompilerParams(use_tc_tiling_on_sc=False),
        out_type=[
            jax.ShapeDtypeStruct((NC, N, 128), _f32),
            jax.ShapeDtypeStruct((NC, N, 16), _f32),
        ],
        scratch_types=[
            pltpu.VMEM((BLK,), jnp.int32),
            pltpu.VMEM((BLK, 128), _f32),
            pltpu.VMEM((BLK, 16), _f32),
            pltpu.VMEM_SHARED((N, 128), _f32),
            pltpu.VMEM_SHARED((N, 16), _f32),
        ],
    )
    def k(ei_h, h_h, pc_h, z128_h, z16_h, hsum_h, psum_h, idx, hv, pv, sh_h, sh_p):
        c = lax.axis_index("c")
        s = lax.axis_index("s")
        base = (s * NC + c) * epw
        r0 = s * rps
        pltpu.sync_copy(z128_h.at[pl.ds(r0, rps)], sh_h.at[pl.ds(r0, rps)])
        pltpu.sync_copy(z16_h.at[pl.ds(r0, rps)], sh_p.at[pl.ds(r0, rps)])
        plsc.subcore_barrier()

        def body(b, carry):
            off = base + b * BLK
            pltpu.sync_copy(ei_h.at[pl.ds(off, BLK)], idx)
            pltpu.sync_copy(h_h.at[pl.ds(off, BLK)], hv)
            pltpu.sync_copy(pc_h.at[pl.ds(off, BLK)], pv)
            pltpu.sync_copy(hv, sh_h.at[idx], add=True)
            pltpu.sync_copy(pv, sh_p.at[idx], add=True)
            return carry

        lax.fori_loop(0, nblk, body, 0)
        plsc.subcore_barrier()
        pltpu.sync_copy(sh_h.at[pl.ds(r0, rps)], hsum_h.at[c, pl.ds(r0, rps)])
        pltpu.sync_copy(sh_p.at[pl.ds(r0, rps)], psum_h.at[c, pl.ds(r0, rps)])

    return k(ei, h, pc, zeros128, zeros16)


def _sc_vmsg(ei, ej, vs0, vs1, vs2, zeros128):
    E = ei.shape[0]
    N = vs0.shape[0]
    epw = E // NW
    nblk = epw // BLK
    rps = N // NS

    @functools.partial(
        pl.kernel,
        mesh=plsc.VectorSubcoreMesh(core_axis_name="c", subcore_axis_name="s"),
        compiler_params=pltpu.CompilerParams(use_tc_tiling_on_sc=False),
        out_type=[jax.ShapeDtypeStruct((NC, 3, N, 128), _f32)],
        scratch_types=[
            pltpu.VMEM((BLK,), jnp.int32),
            pltpu.VMEM((BLK,), jnp.int32),
            pltpu.VMEM((BLK, 128), _f32),
            pltpu.VMEM_SHARED((N, 128), _f32),
            pltpu.SemaphoreType.DMA,
        ],
    )
    def k(ei_h, ej_h, vs0_h, vs1_h, vs2_h, z128_h, mv_h, idxi, idxj, rows, sh, sem):
        c = lax.axis_index("c")
        s = lax.axis_index("s")
        base = (s * NC + c) * epw
        r0 = s * rps
        for t, vs_h in enumerate((vs0_h, vs1_h, vs2_h)):
            pltpu.sync_copy(z128_h.at[pl.ds(r0, rps)], sh.at[pl.ds(r0, rps)])
            plsc.subcore_barrier()

            def body(b, carry):
                off = base + b * BLK
                pltpu.sync_copy(ei_h.at[pl.ds(off, BLK)], idxi)
                pltpu.sync_copy(ej_h.at[pl.ds(off, BLK)], idxj)
                pltpu.async_copy(vs_h.at[idxj], rows, sem).wait()
                pltpu.sync_copy(rows, sh.at[idxi], add=True)
                return carry

            lax.fori_loop(0, nblk, body, 0)
            plsc.subcore_barrier()
            pltpu.sync_copy(sh.at[pl.ds(r0, rps)], mv_h.at[c, t, pl.ds(r0, rps)])
            if t < 2:
                plsc.subcore_barrier()

    return k(ei, ej, vs0, vs1, vs2, zeros128)


def _tc_fin(s, vT, pos16, hsumP, psumP, mvP, e_w2, e_b2, sm_w1a, sm_w1b,
            sm_b1, sm_w2, sm_b2, g_w, g_b):
    N, H = s.shape
    BN = 2000

    def body(s_ref, vT_ref, pos_ref, hp_ref, pp_ref, mv_ref, ew2_ref, eb2_ref,
             w1a_ref, w1b_ref, b1_ref, w2_ref, b2_ref, gw_ref, gb_ref,
             snew_ref, vnew_ref, pnew_ref):
        sb = s_ref[...]
        hsum = hp_ref[0] + hp_ref[1]
        ps = pp_ref[0] + pp_ref[1]
        cnt = ps[:, 3:4]
        m_s = jnp.dot(hsum, ew2_ref[...], preferred_element_type=_f32) + cnt * eb2_ref[...]
        x = (jnp.dot(sb, w1a_ref[...], preferred_element_type=_f32)
             + jnp.dot(m_s, w1b_ref[...], preferred_element_type=_f32) + b1_ref[...])
        s_up = jnp.dot(_silu(x), w2_ref[...], preferred_element_type=_f32) + b2_ref[...]
        s_new = sb + s_up
        snew_ref[...] = s_new
        g = jax.nn.sigmoid(jnp.dot(s_new, gw_ref[...], preferred_element_type=_f32)
                           + gb_ref[...])
        mv = mv_ref[0] + mv_ref[1]
        vnew_ref[...] = vT_ref[...] + mv * g[None]
        sums3 = ps[:, 0:3]
        delta = jnp.where(cnt > 0, sums3 / jnp.maximum(cnt, 1.0), 0.0)
        delta = jnp.clip(delta, -1.0, 1.0)
        p3 = pos_ref[...][:, 0:3] + delta * 1e-05
        pnew_ref[...] = jnp.concatenate([p3, jnp.zeros((BN, 13), _f32)], axis=1)

    full = lambda shape: pl.BlockSpec(shape, lambda n: (0,) * len(shape))
    return pl.pallas_call(
        body,
        grid=(N // BN,),
        in_specs=[
            pl.BlockSpec((BN, H), lambda n: (n, 0)),
            pl.BlockSpec((3, BN, H), lambda n: (0, n, 0)),
            pl.BlockSpec((BN, 16), lambda n: (n, 0)),
            pl.BlockSpec((NC, BN, H), lambda n: (0, n, 0)),
            pl.BlockSpec((NC, BN, 16), lambda n: (0, n, 0)),
            pl.BlockSpec((NC, 3, BN, H), lambda n: (0, 0, n, 0)),
            full((H, H)), full((1, H)), full((H, H)), full((H, H)),
            full((1, H)), full((H, H)), full((1, H)), full((H, H)), full((1, H)),
        ],
        out_specs=[
            pl.BlockSpec((BN, H), lambda n: (n, 0)),
            pl.BlockSpec((3, BN, H), lambda n: (0, n, 0)),
            pl.BlockSpec((BN, 16), lambda n: (n, 0)),
        ],
        out_shape=[
            jax.ShapeDtypeStruct((N, H), _f32),
            jax.ShapeDtypeStruct((3, N, H), _f32),
            jax.ShapeDtypeStruct((N, 16), _f32),
        ],
    )(s, vT, pos16, hsumP, psumP, mvP, e_w2, e_b2, sm_w1a, sm_w1b, sm_b1,
      sm_w2, sm_b2, g_w, g_b)


def kernel(s, v, pos, edges, edge_w1, edge_b1, edge_w2, edge_b2, posmlp_w1,
           posmlp_b1, posmlp_w2, posmlp_b2, vpos_w, vmsg_w1, vmsg_b1, vmsg_w2,
           vmsg_b2, smlp_w1, smlp_b1, smlp_w2, smlp_b2, gate_w, gate_b):
    N, H = s.shape
    ei = edges[0].astype(jnp.int32)
    ej = edges[1].astype(jnp.int32)

    w1a = edge_w1[:H]
    w1b = edge_w1[H:2 * H]
    w1c = edge_w1[2 * H].reshape(1, H)
    Wep = edge_w2 @ posmlp_w1
    bep = (edge_b2 @ posmlp_w1 + posmlp_b1).reshape(1, H)
    pb2 = posmlp_b2.reshape(1, 1)

    vT = jnp.transpose(v, (2, 0, 1))
    pos16 = jnp.pad(pos, ((0, 0), (0, 13)))
    zeros128 = jnp.zeros((N, 128), _f32)
    zeros16 = jnp.zeros((N, 16), _f32)

    tabI, tabJ, vs = _tc_pre(s, vT, pos16, w1a, w1b, edge_b1.reshape(1, H),
                             vmsg_w1, vmsg_b1.reshape(1, H), vmsg_w2,
                             vmsg_b2.reshape(1, H), vpos_w.reshape(1, H))
    gI, gJ = _sc_gather(ei, ej, tabI, tabJ)
    h, pc = _tc_edge(gI, gJ, w1c, Wep, bep, posmlp_w2, pb2)
    hsumP, psumP = _sc_scatter_hp(ei, h, pc, zeros128, zeros16)
    mvP = _sc_vmsg(ei, ej, vs[0], vs[1], vs[2], zeros128)
    if isinstance(mvP, (list, tuple)):
        mvP = mvP[0]

    s_new, vnT, pos_n = _tc_fin(
        s, vT, pos16, hsumP, psumP, mvP, edge_w2, edge_b2.reshape(1, H),
        smlp_w1[:H], smlp_w1[H:], smlp_b1.reshape(1, H), smlp_w2,
        smlp_b2.reshape(1, H), gate_w, gate_b.reshape(1, H))

    v_new = jnp.transpose(vnT, (1, 2, 0))
    pos_new = pos_n[:, :3]
    return (s_new, v_new, pos_new)
